# baseline (speedup 1.0000x reference)
import functools

import jax
import jax.numpy as jnp
from jax import lax
from jax.experimental import pallas as pl
from jax.experimental.pallas import tpu as pltpu
from jax.experimental.pallas import tpu_sc as plsc

N = 50000
C = 64
E = 800000

NSC = 2
NTILE = 16
HALF = N // NSC
ACC_ROWS = HALF + 8
DUMMY = HALF
K = 80
CPB = 16
B = K * CPB
EPT = 49920
NBLK_LO = EPT // B
NBLK_HI = (E - (NTILE - 1) * EPT) // B
NBUF = 4
VPC = K // 16

STRIPE = 1568
LAST_STRIPE = HALF - (NTILE - 1) * STRIPE


def _sc_body(x_hbm, src_hbm, tgt_hbm, out_hbm,
             sbufs0, sbuft0, sbufs1, sbuft1,
             cbufs, cbuft,
             rows0, rows1, rows2, rows3,
             sidx0, sidx1, sidx2, sidx3,
             acc,
             semg0, semg1, semg2, semg3,
             sems0, sems1, sems2, sems3,
             semi0, semi1):
    sc = lax.axis_index("c")
    tile = lax.axis_index("s")
    node_base = sc * HALF
    ebase = tile * EPT
    nblk = jnp.where(tile == NTILE - 1, NBLK_HI, NBLK_LO)

    sbufs = (sbufs0, sbufs1)
    sbuft = (sbuft0, sbuft1)
    semi = (semi0, semi1)
    rows = (rows0, rows1, rows2, rows3)
    sidx = (sidx0, sidx1, sidx2, sidx3)
    semg = (semg0, semg1, semg2, semg3)
    sems = (sems0, sems1, sems2, sems3)

    def _stage_start(blk, par):
        pltpu.async_copy(src_hbm.at[pl.ds(ebase + blk * B, B)],
                         sbufs[par], semi[par])
        pltpu.async_copy(tgt_hbm.at[pl.ds(ebase + blk * B, B)],
                         sbuft[par], semi[par])

    def _stage_wait(blk, par):
        pltpu.make_async_copy(src_hbm.at[pl.ds(ebase + blk * B, B)],
                              sbufs[par], semi[par]).wait()
        pltpu.make_async_copy(tgt_hbm.at[pl.ds(ebase + blk * B, B)],
                              sbuft[par], semi[par]).wait()

    def _gather_start(j, b):
        pltpu.async_copy(x_hbm.at[cbufs.at[pl.ds(j * K, K)]],
                         rows[b], semg[b])

    def _gather_wait(j, b):
        pltpu.make_async_copy(x_hbm.at[cbufs.at[pl.ds(j * K, K)]],
                              rows[b], semg[b]).wait()

    def _scatter_start(b):
        pltpu.async_copy(rows[b], acc.at[sidx[b]], sems[b], add=True)

    def _scatter_wait(b):
        pltpu.make_async_copy(rows[b], acc.at[sidx[b]], sems[b]).wait()

    def _compact(par):
        def _cbody(v, off):
            s = sbufs[par][pl.ds(v * 16, 16)]
            t = sbuft[par][pl.ds(v * 16, 16)]
            lo = t - node_base
            ok = (lo >= 0) & (lo < HALF)
            oki = jnp.where(ok, 1, 0)
            pos = off + plsc.cumsum(oki) - oki
            plsc.store_scatter(cbufs, [pos], s, mask=ok)
            plsc.store_scatter(cbuft, [pos], lo, mask=ok)
            return off + plsc.all_reduce_population_count(ok)

        off = lax.fori_loop(0, B // 16, _cbody,
                            jnp.zeros((16,), jnp.int32))
        cnt = jnp.max(off, axis=0)
        padded = ((cnt + K - 1) // K) * K
        lane = lax.iota(jnp.int32, 16)
        zeros16 = jnp.zeros((16,), jnp.int32)
        dummy16 = jnp.full((16,), DUMMY, jnp.int32)
        for u in range(K // 16):
            posp = off + lane + u * 16
            okp = posp < padded
            plsc.store_scatter(cbufs, [posp], zeros16, mask=okp)
            plsc.store_scatter(cbuft, [posp], dummy16, mask=okp)
        return padded // K

    _stage_start(0, 0)

    @pl.when(tile < NTILE - 1)
    def _():
        r0 = tile * STRIPE
        pltpu.sync_copy(x_hbm.at[pl.ds(node_base + r0, STRIPE)],
                        acc.at[pl.ds(r0, STRIPE)])

    @pl.when(tile == NTILE - 1)
    def _():
        r0 = (NTILE - 1) * STRIPE
        pltpu.sync_copy(x_hbm.at[pl.ds(node_base + r0, LAST_STRIPE)],
                        acc.at[pl.ds(r0, LAST_STRIPE)])

    plsc.subcore_barrier()

    def _run_block(par):
        nch = _compact(par)

        for jj in range(NBUF - 1):
            @pl.when(jj < nch)
            def _():
                _gather_start(jj, jj)

        def _grp(gidx, c):
            for u in range(NBUF):
                j = gidx * NBUF + u
                bn = (u + NBUF - 1) % NBUF

                @pl.when(j < nch)
                def _():
                    _gather_wait(j, u)
                    for v in range(VPC):
                        sidx[u][pl.ds(v * 16, 16)] = (
                            cbuft[pl.ds(j * K + v * 16, 16)])
                    _scatter_start(u)

                    @pl.when(j >= 1)
                    def _():
                        _scatter_wait(bn)

                    @pl.when(j + NBUF - 1 < nch)
                    def _():
                        _gather_start(j + NBUF - 1, bn)
            return c

        lax.fori_loop(0, CPB // NBUF, _grp, 0)

        @pl.when(nch > 0)
        def _():
            for b in range(NBUF):
                @pl.when((nch - 1) % NBUF == b)
                def _():
                    _scatter_wait(b)

    def _block_pair(p, carry):
        for par in range(2):
            blk = p * 2 + par

            @pl.when(blk < nblk)
            def _():
                _stage_wait(blk, par)

                @pl.when(blk + 1 < nblk)
                def _():
                    _stage_start(blk + 1, 1 - par)

                _run_block(par)
        return carry

    lax.fori_loop(0, (NBLK_HI + 1) // 2, _block_pair, 0)

    plsc.subcore_barrier()

    @pl.when(tile < NTILE - 1)
    def _():
        r0 = tile * STRIPE
        pltpu.sync_copy(acc.at[pl.ds(r0, STRIPE)],
                        out_hbm.at[pl.ds(node_base + r0, STRIPE)])

    @pl.when(tile == NTILE - 1)
    def _():
        r0 = (NTILE - 1) * STRIPE
        pltpu.sync_copy(acc.at[pl.ds(r0, LAST_STRIPE)],
                        out_hbm.at[pl.ds(node_base + r0, LAST_STRIPE)])


_sc_aggregate = functools.partial(
    pl.kernel,
    out_type=jax.ShapeDtypeStruct((N, C), jnp.float32),
    mesh=plsc.VectorSubcoreMesh(core_axis_name="c", subcore_axis_name="s"),
    scratch_types=(
        [pltpu.VMEM((B,), jnp.int32)] * 4
        + [pltpu.VMEM((B,), jnp.int32)] * 2
        + [pltpu.VMEM((K, C), jnp.float32)] * 4
        + [pltpu.VMEM((K,), jnp.int32)] * 4
        + [pltpu.VMEM_SHARED((ACC_ROWS, C), jnp.float32)]
        + [pltpu.SemaphoreType.DMA] * 10
    ),
    compiler_params=pltpu.CompilerParams(use_tc_tiling_on_sc=False,
                                         needs_layout_passes=False),
)(_sc_body)


_R = 2000


def _tc_body(agg_ref, norm_ref, w_ref, out_ref):
    h = norm_ref[...] * agg_ref[...]
    out_ref[...] = lax.dot_general(
        h, w_ref[...], (((1,), (0,)), ((), ())),
        precision=lax.Precision.HIGHEST,
        preferred_element_type=jnp.float32)


def _tc_matmul(agg, norm, w):
    return pl.pallas_call(
        _tc_body,
        grid=(N // _R,),
        in_specs=[
            pl.BlockSpec((_R, C), lambda i: (i, 0)),
            pl.BlockSpec((_R, 1), lambda i: (i, 0)),
            pl.BlockSpec((C, C), lambda i: (0, 0)),
        ],
        out_specs=pl.BlockSpec((_R, C), lambda i: (i, 0)),
        out_shape=jax.ShapeDtypeStruct((N, C), jnp.float32),
    )(agg, norm, w)


def kernel(x, sources, targets, norm, W):
    src = sources.astype(jnp.int32)
    tgt = targets.astype(jnp.int32)
    agg = _sc_aggregate(x, src, tgt)
    return _tc_matmul(agg, norm, W)


# speedup vs baseline: 2.3564x; 2.3564x over previous
import functools

import jax
import jax.numpy as jnp
from jax import lax
from jax.experimental import pallas as pl
from jax.experimental.pallas import tpu as pltpu
from jax.experimental.pallas import tpu_sc as plsc

N = 50000
C = 64
E = 800000

NSC = 2
NTILE = 16
CH = C // NSC
K = 80
CPB = 16
B = K * CPB
EPT = 49920
NBLK_LO = EPT // B
NBLK_HI = (E - (NTILE - 1) * EPT) // B
NBUF = 4
VPC = K // 16

STRIPE = 3128
LAST_STRIPE = N - (NTILE - 1) * STRIPE


def _sc_body(xs_hbm, srcab_hbm, tgt_hbm, out_hbm,
             sbufs0, sbuft0, sbufs1, sbuft1,
             rows0, rows1, rows2, rows3,
             sidx0, sidx1, sidx2, sidx3,
             acc,
             semg0, semg1, semg2, semg3,
             sems0, sems1, sems2, sems3,
             semi0, semi1):
    sc = lax.axis_index("c")
    tile = lax.axis_index("s")
    xbase = sc * N
    ebase = tile * EPT
    sbase = sc * E + ebase
    nblk = jnp.where(tile == NTILE - 1, NBLK_HI, NBLK_LO)

    sbufs = (sbufs0, sbufs1)
    sbuft = (sbuft0, sbuft1)
    semi = (semi0, semi1)
    rows = (rows0, rows1, rows2, rows3)
    sidx = (sidx0, sidx1, sidx2, sidx3)
    semg = (semg0, semg1, semg2, semg3)
    sems = (sems0, sems1, sems2, sems3)

    def _stage_start(blk, par):
        pltpu.async_copy(srcab_hbm.at[pl.ds(sbase + blk * B, B)],
                         sbufs[par], semi[par])
        pltpu.async_copy(tgt_hbm.at[pl.ds(ebase + blk * B, B)],
                         sbuft[par], semi[par])

    def _stage_wait(blk, par):
        pltpu.make_async_copy(srcab_hbm.at[pl.ds(sbase + blk * B, B)],
                              sbufs[par], semi[par]).wait()
        pltpu.make_async_copy(tgt_hbm.at[pl.ds(ebase + blk * B, B)],
                              sbuft[par], semi[par]).wait()

    def _gather_start(j, b, par):
        pltpu.async_copy(xs_hbm.at[sbufs[par].at[pl.ds(j * K, K)]],
                         rows[b], semg[b])

    def _gather_wait(j, b, par):
        pltpu.make_async_copy(xs_hbm.at[sbufs[par].at[pl.ds(j * K, K)]],
                              rows[b], semg[b]).wait()

    def _scatter_start(b):
        pltpu.async_copy(rows[b], acc.at[sidx[b]], sems[b], add=True)

    def _scatter_wait(b):
        pltpu.make_async_copy(rows[b], acc.at[sidx[b]], sems[b]).wait()

    _stage_start(0, 0)

    @pl.when(tile < NTILE - 1)
    def _():
        r0 = tile * STRIPE
        pltpu.sync_copy(xs_hbm.at[pl.ds(xbase + r0, STRIPE)],
                        acc.at[pl.ds(r0, STRIPE)])

    @pl.when(tile == NTILE - 1)
    def _():
        r0 = (NTILE - 1) * STRIPE
        pltpu.sync_copy(xs_hbm.at[pl.ds(xbase + r0, LAST_STRIPE)],
                        acc.at[pl.ds(r0, LAST_STRIPE)])

    plsc.subcore_barrier()

    def _run_block(par):
        for jj in range(NBUF - 1):
            _gather_start(jj, jj, par)

        def _grp(gidx, c):
            for u in range(NBUF):
                j = gidx * NBUF + u
                bn = (u + NBUF - 1) % NBUF
                _gather_wait(j, u, par)
                for v in range(VPC):
                    sidx[u][pl.ds(v * 16, 16)] = (
                        sbuft[par][pl.ds(j * K + v * 16, 16)])
                _scatter_start(u)
                if u == 0:
                    @pl.when(gidx > 0)
                    def _():
                        _scatter_wait(bn)
                else:
                    _scatter_wait(bn)

                @pl.when(j < CPB - NBUF + 1)
                def _():
                    _gather_start(j + NBUF - 1, bn, par)
            return c

        lax.fori_loop(0, CPB // NBUF, _grp, 0)
        _scatter_wait(NBUF - 1)

    def _block_pair(p, carry):
        for par in range(2):
            blk = p * 2 + par

            @pl.when(blk < nblk)
            def _():
                _stage_wait(blk, par)

                @pl.when(blk + 1 < nblk)
                def _():
                    _stage_start(blk + 1, 1 - par)

                _run_block(par)
        return carry

    lax.fori_loop(0, (NBLK_HI + 1) // 2, _block_pair, 0)

    plsc.subcore_barrier()

    @pl.when(tile < NTILE - 1)
    def _():
        r0 = tile * STRIPE
        pltpu.sync_copy(acc.at[pl.ds(r0, STRIPE)],
                        out_hbm.at[pl.ds(xbase + r0, STRIPE)])

    @pl.when(tile == NTILE - 1)
    def _():
        r0 = (NTILE - 1) * STRIPE
        pltpu.sync_copy(acc.at[pl.ds(r0, LAST_STRIPE)],
                        out_hbm.at[pl.ds(xbase + r0, LAST_STRIPE)])


_sc_aggregate = functools.partial(
    pl.kernel,
    out_type=jax.ShapeDtypeStruct((NSC * N, CH), jnp.float32),
    mesh=plsc.VectorSubcoreMesh(core_axis_name="c", subcore_axis_name="s"),
    scratch_types=(
        [pltpu.VMEM((B,), jnp.int32)] * 4
        + [pltpu.VMEM((K, CH), jnp.float32)] * 4
        + [pltpu.VMEM((K,), jnp.int32)] * 4
        + [pltpu.VMEM_SHARED((N, CH), jnp.float32)]
        + [pltpu.SemaphoreType.DMA] * 10
    ),
    compiler_params=pltpu.CompilerParams(use_tc_tiling_on_sc=False,
                                         needs_layout_passes=False),
)(_sc_body)


_R = 2000
_GN = N // _R
_EB = E // _GN


def _pre_body(x_ref, src_ref, xs_ref, srcab_ref):
    half = pl.program_id(0) // _GN
    xv = x_ref[...]
    xs_ref[...] = jnp.where(half == 0, xv[:, :CH], xv[:, CH:])
    srcab_ref[...] = src_ref[...] + half * N


def _tc_pre(x, src3):
    xs, srcab3 = pl.pallas_call(
        _pre_body,
        grid=(2 * _GN,),
        in_specs=[
            pl.BlockSpec((_R, C), lambda g: (g % _GN, 0)),
            pl.BlockSpec((1, 1, _EB), lambda g: (g % _GN, 0, 0)),
        ],
        out_specs=[
            pl.BlockSpec((_R, CH), lambda g: (g, 0)),
            pl.BlockSpec((1, 1, _EB), lambda g: (g, 0, 0)),
        ],
        out_shape=[
            jax.ShapeDtypeStruct((NSC * N, CH), jnp.float32),
            jax.ShapeDtypeStruct((2 * _GN, 1, _EB), jnp.int32),
        ],
    )(x, src3)
    return xs, srcab3.reshape(NSC * E)


def _post_body(a0_ref, a1_ref, norm_ref, w0_ref, w1_ref, out_ref):
    dn = (((1,), (0,)), ((), ()))
    acc = (lax.dot_general(a0_ref[...], w0_ref[...], dn,
                           precision=lax.Precision.HIGHEST,
                           preferred_element_type=jnp.float32)
           + lax.dot_general(a1_ref[...], w1_ref[...], dn,
                             precision=lax.Precision.HIGHEST,
                             preferred_element_type=jnp.float32))
    out_ref[...] = norm_ref[...] * acc


def _tc_post(agg, norm, w):
    return pl.pallas_call(
        _post_body,
        grid=(_GN,),
        in_specs=[
            pl.BlockSpec((_R, CH), lambda i: (i, 0)),
            pl.BlockSpec((_R, CH), lambda i: (i + _GN, 0)),
            pl.BlockSpec((_R, 1), lambda i: (i, 0)),
            pl.BlockSpec((CH, C), lambda i: (0, 0)),
            pl.BlockSpec((CH, C), lambda i: (1, 0)),
        ],
        out_specs=pl.BlockSpec((_R, C), lambda i: (i, 0)),
        out_shape=jax.ShapeDtypeStruct((N, C), jnp.float32),
    )(agg, agg, norm, w, w)


def kernel(x, sources, targets, norm, W):
    src3 = sources.astype(jnp.int32).reshape(_GN, 1, _EB)
    tgt = targets.astype(jnp.int32)
    xs, srcab = _tc_pre(x, src3)
    agg = _sc_aggregate(xs, srcab, tgt)
    return _tc_post(agg, norm, W)


# speedup vs baseline: 3.0686x; 1.3022x over previous
import functools

import jax
import jax.numpy as jnp
from jax import lax
from jax.experimental import pallas as pl
from jax.experimental.pallas import tpu as pltpu
from jax.experimental.pallas import tpu_sc as plsc

N = 50000
C = 64
E = 800000

NSC = 2
NTILE = 16
CH = C // NSC
K = 80
CPB = 16
B = K * CPB
EPT = 49920
NBLK_LO = EPT // B
NBLK_HI = (E - (NTILE - 1) * EPT) // B
NBUF = 4
VPC = K // 16

STRIPE = 3128
LAST_STRIPE = N - (NTILE - 1) * STRIPE


def _sc_body(x2_hbm, src_hbm, tgt_hbm, zin_hbm, out_hbm,
             sbufs0, sbuft0, sbufs1, sbuft1,
             rows0, rows1, rows2, rows3,
             sidx0, sidx1, sidx2, sidx3,
             acc,
             semg0, semg1, semg2, semg3,
             sems0, sems1, sems2, sems3,
             semi0, semi1):
    sc = lax.axis_index("c")
    tile = lax.axis_index("s")
    xbase = sc * N
    ebase = tile * EPT
    nblk = jnp.where(tile == NTILE - 1, NBLK_HI, NBLK_LO)

    sbufs = (sbufs0, sbufs1)
    sbuft = (sbuft0, sbuft1)
    semi = (semi0, semi1)
    rows = (rows0, rows1, rows2, rows3)
    sidx = (sidx0, sidx1, sidx2, sidx3)
    semg = (semg0, semg1, semg2, semg3)
    sems = (sems0, sems1, sems2, sems3)

    def _stage_start(blk, par):
        pltpu.async_copy(src_hbm.at[pl.ds(ebase + blk * B, B)],
                         sbufs[par], semi[par])
        pltpu.async_copy(tgt_hbm.at[pl.ds(ebase + blk * B, B)],
                         sbuft[par], semi[par])

    def _stage_wait(blk, par):
        pltpu.make_async_copy(src_hbm.at[pl.ds(ebase + blk * B, B)],
                              sbufs[par], semi[par]).wait()
        pltpu.make_async_copy(tgt_hbm.at[pl.ds(ebase + blk * B, B)],
                              sbuft[par], semi[par]).wait()

    def _gather_start(j, b, par):
        pltpu.async_copy(x2_hbm.at[sbufs[par].at[pl.ds(j * K, K)]],
                         rows[b], semg[b])

    def _gather_wait(j, b, par):
        pltpu.make_async_copy(x2_hbm.at[sbufs[par].at[pl.ds(j * K, K)]],
                              rows[b], semg[b]).wait()

    def _scatter_start(b):
        pltpu.async_copy(rows[b], acc.at[sidx[b]], sems[b], add=True)

    def _scatter_wait(b):
        pltpu.make_async_copy(rows[b], acc.at[sidx[b]], sems[b]).wait()

    _stage_start(0, 0)

    @pl.when(tile < NTILE - 1)
    def _():
        r0 = tile * STRIPE
        pltpu.sync_copy(zin_hbm, acc.at[pl.ds(r0, STRIPE)])

    @pl.when(tile == NTILE - 1)
    def _():
        r0 = (NTILE - 1) * STRIPE
        pltpu.sync_copy(zin_hbm.at[pl.ds(0, LAST_STRIPE)],
                        acc.at[pl.ds(r0, LAST_STRIPE)])

    plsc.subcore_barrier()

    def _bias(par):
        def _bb(v, c):
            sl = pl.ds(v * 16, 16)
            sbufs[par][sl] = sbufs[par][sl] * 2 + sc
            return c

        lax.fori_loop(0, B // 16, _bb, 0, unroll=4)

    def _run_block(par):
        for jj in range(NBUF - 1):
            _gather_start(jj, jj, par)

        def _grp(gidx, c):
            for u in range(NBUF):
                j = gidx * NBUF + u
                bn = (u + NBUF - 1) % NBUF
                _gather_wait(j, u, par)
                for v in range(VPC):
                    sidx[u][pl.ds(v * 16, 16)] = (
                        sbuft[par][pl.ds(j * K + v * 16, 16)])
                _scatter_start(u)
                if u == 0:
                    @pl.when(gidx > 0)
                    def _():
                        _scatter_wait(bn)
                else:
                    _scatter_wait(bn)

                @pl.when(j < CPB - NBUF + 1)
                def _():
                    _gather_start(j + NBUF - 1, bn, par)
            return c

        lax.fori_loop(0, CPB // NBUF, _grp, 0)
        _scatter_wait(NBUF - 1)

    def _block_pair(p, carry):
        for par in range(2):
            blk = p * 2 + par

            @pl.when(blk < nblk)
            def _():
                _stage_wait(blk, par)

                @pl.when(blk + 1 < nblk)
                def _():
                    _stage_start(blk + 1, 1 - par)

                _bias(par)
                _run_block(par)
        return carry

    lax.fori_loop(0, (NBLK_HI + 1) // 2, _block_pair, 0)

    plsc.subcore_barrier()

    @pl.when(tile < NTILE - 1)
    def _():
        r0 = tile * STRIPE
        pltpu.sync_copy(acc.at[pl.ds(r0, STRIPE)],
                        out_hbm.at[pl.ds(xbase + r0, STRIPE)])

    @pl.when(tile == NTILE - 1)
    def _():
        r0 = (NTILE - 1) * STRIPE
        pltpu.sync_copy(acc.at[pl.ds(r0, LAST_STRIPE)],
                        out_hbm.at[pl.ds(xbase + r0, LAST_STRIPE)])


_sc_aggregate = functools.partial(
    pl.kernel,
    out_type=jax.ShapeDtypeStruct((NSC * N, CH), jnp.float32),
    mesh=plsc.VectorSubcoreMesh(core_axis_name="c", subcore_axis_name="s"),
    scratch_types=(
        [pltpu.VMEM((B,), jnp.int32)] * 4
        + [pltpu.VMEM((K, CH), jnp.float32)] * 4
        + [pltpu.VMEM((K,), jnp.int32)] * 4
        + [pltpu.VMEM_SHARED((N, CH), jnp.float32)]
        + [pltpu.SemaphoreType.DMA] * 10
    ),
    compiler_params=pltpu.CompilerParams(use_tc_tiling_on_sc=False,
                                         needs_layout_passes=False),
)(_sc_body)


_R = 2000
_GN = N // _R


def _post_body(x_ref, a0_ref, a1_ref, norm_ref, w0_ref, w1_ref, out_ref):
    dn = (((1,), (0,)), ((), ()))
    xv = x_ref[...]
    h0 = xv[:, :CH] + a0_ref[...]
    h1 = xv[:, CH:] + a1_ref[...]
    acc = (lax.dot_general(h0, w0_ref[...], dn,
                           precision=lax.Precision.HIGHEST,
                           preferred_element_type=jnp.float32)
           + lax.dot_general(h1, w1_ref[...], dn,
                             precision=lax.Precision.HIGHEST,
                             preferred_element_type=jnp.float32))
    out_ref[...] = norm_ref[...] * acc


def _tc_post(x, agg, norm, w):
    return pl.pallas_call(
        _post_body,
        grid=(_GN,),
        in_specs=[
            pl.BlockSpec((_R, C), lambda i: (i, 0)),
            pl.BlockSpec((_R, CH), lambda i: (i, 0)),
            pl.BlockSpec((_R, CH), lambda i: (i + _GN, 0)),
            pl.BlockSpec((_R, 1), lambda i: (i, 0)),
            pl.BlockSpec((CH, C), lambda i: (0, 0)),
            pl.BlockSpec((CH, C), lambda i: (1, 0)),
        ],
        out_specs=pl.BlockSpec((_R, C), lambda i: (i, 0)),
        out_shape=jax.ShapeDtypeStruct((N, C), jnp.float32),
    )(x, agg, agg, norm, w, w)


def kernel(x, sources, targets, norm, W):
    src = sources.astype(jnp.int32)
    tgt = targets.astype(jnp.int32)
    x2 = x.reshape(NSC * N, CH)
    zin = jnp.zeros((STRIPE, CH), jnp.float32)
    agg = _sc_aggregate(x2, src, tgt, zin)
    return _tc_post(x, agg, norm, W)


# speedup vs baseline: 3.4928x; 1.1382x over previous
import functools

import jax
import jax.numpy as jnp
from jax import lax
from jax.experimental import pallas as pl
from jax.experimental.pallas import tpu as pltpu
from jax.experimental.pallas import tpu_sc as plsc

N = 50000
C = 64
E = 800000

NSC = 2
NTILE = 16
CH = C // NSC
K = 80
CPB = 16
B = K * CPB
EPT = 49920
NBLK_LO = EPT // B
NBLK_HI = (E - (NTILE - 1) * EPT) // B
NBUF = 4
VPC = K // 16

STRIPE = 3120
LAST_STRIPE = N - (NTILE - 1) * STRIPE
NCI_LO = STRIPE // K
NCI_HI = LAST_STRIPE // K
AGG_HALF = 52000


def _sc_body(x2_hbm, src_hbm, tgt_hbm, out_hbm,
             sbufs0, sbuft0, sbufs1, sbuft1,
             rows0, rows1, rows2, rows3,
             sidx0, sidx1, sidx2, sidx3,
             acc,
             semg0, semg1, semg2, semg3,
             sems0, sems1, sems2, sems3,
             semi0, semi1):
    sc = lax.axis_index("c")
    tile = lax.axis_index("s")
    xbase = sc * AGG_HALF
    ebase = tile * EPT
    nblk = jnp.where(tile == NTILE - 1, NBLK_HI, NBLK_LO)

    sbufs = (sbufs0, sbufs1)
    sbuft = (sbuft0, sbuft1)
    semi = (semi0, semi1)
    rows = (rows0, rows1, rows2, rows3)
    sidx = (sidx0, sidx1, sidx2, sidx3)
    semg = (semg0, semg1, semg2, semg3)
    sems = (sems0, sems1, sems2, sems3)

    def _stage_start(blk, par):
        pltpu.async_copy(src_hbm.at[pl.ds(ebase + blk * B, B)],
                         sbufs[par], semi[par])
        pltpu.async_copy(tgt_hbm.at[pl.ds(ebase + blk * B, B)],
                         sbuft[par], semi[par])

    def _stage_wait(blk, par):
        pltpu.make_async_copy(src_hbm.at[pl.ds(ebase + blk * B, B)],
                              sbufs[par], semi[par]).wait()
        pltpu.make_async_copy(tgt_hbm.at[pl.ds(ebase + blk * B, B)],
                              sbuft[par], semi[par]).wait()

    def _gather_start(j, b, par):
        pltpu.async_copy(x2_hbm.at[sbufs[par].at[pl.ds(j * K, K)]],
                         rows[b], semg[b])

    def _gather_wait(j, b, par):
        pltpu.make_async_copy(x2_hbm.at[sbufs[par].at[pl.ds(j * K, K)]],
                              rows[b], semg[b]).wait()

    def _scatter_start(b):
        pltpu.async_copy(rows[b], acc.at[sidx[b]], sems[b], add=True)

    def _scatter_wait(b):
        pltpu.make_async_copy(rows[b], acc.at[sidx[b]], sems[b]).wait()

    _stage_start(0, 0)

    r0 = tile * STRIPE
    nci = jnp.where(tile == NTILE - 1, NCI_HI, NCI_LO)
    lane = lax.iota(jnp.int32, 16)

    def _ibuild(c, b):
        base = (r0 + c * K) * 2 + sc
        for v in range(VPC):
            sidx[b][pl.ds(v * 16, 16)] = base + (lane + v * 16) * 2

    def _igather_start(b):
        pltpu.async_copy(x2_hbm.at[sidx[b]], rows[b], semg[b])

    def _igather_wait(b):
        pltpu.make_async_copy(x2_hbm.at[sidx[b]], rows[b], semg[b]).wait()

    _ibuild(0, 0)
    _igather_start(0)

    def _ipair(p, carry):
        for par in range(2):
            c = p * 2 + par

            @pl.when(c < nci)
            def _():
                @pl.when(c + 1 < nci)
                def _():
                    _ibuild(c + 1, 1 - par)
                    _igather_start(1 - par)

                _igather_wait(par)
                pltpu.sync_copy(rows[par], acc.at[pl.ds(r0 + c * K, K)])
        return carry

    lax.fori_loop(0, (NCI_HI + 1) // 2, _ipair, 0)

    plsc.subcore_barrier()

    def _bias(par):
        def _bb(v, c):
            sl = pl.ds(v * 16, 16)
            sbufs[par][sl] = sbufs[par][sl] * 2 + sc
            return c

        lax.fori_loop(0, B // 16, _bb, 0, unroll=4)

    def _run_block(par):
        for jj in range(NBUF - 1):
            _gather_start(jj, jj, par)

        def _grp(gidx, c):
            for u in range(NBUF):
                j = gidx * NBUF + u
                bn = (u + NBUF - 1) % NBUF
                _gather_wait(j, u, par)
                for v in range(VPC):
                    sidx[u][pl.ds(v * 16, 16)] = (
                        sbuft[par][pl.ds(j * K + v * 16, 16)])
                _scatter_start(u)
                if u == 0:
                    @pl.when(gidx > 0)
                    def _():
                        _scatter_wait(bn)
                else:
                    _scatter_wait(bn)

                @pl.when(j < CPB - NBUF + 1)
                def _():
                    _gather_start(j + NBUF - 1, bn, par)
            return c

        lax.fori_loop(0, CPB // NBUF, _grp, 0)
        _scatter_wait(NBUF - 1)

    def _block_pair(p, carry):
        for par in range(2):
            blk = p * 2 + par

            @pl.when(blk < nblk)
            def _():
                _stage_wait(blk, par)

                @pl.when(blk + 1 < nblk)
                def _():
                    _stage_start(blk + 1, 1 - par)

                _bias(par)
                _run_block(par)
        return carry

    lax.fori_loop(0, (NBLK_HI + 1) // 2, _block_pair, 0)

    plsc.subcore_barrier()

    @pl.when(tile < NTILE - 1)
    def _():
        r0 = tile * STRIPE
        pltpu.sync_copy(acc.at[pl.ds(r0, STRIPE)],
                        out_hbm.at[pl.ds(xbase + r0, STRIPE)])

    @pl.when(tile == NTILE - 1)
    def _():
        r0 = (NTILE - 1) * STRIPE
        pltpu.sync_copy(acc.at[pl.ds(r0, LAST_STRIPE)],
                        out_hbm.at[pl.ds(xbase + r0, LAST_STRIPE)])


_sc_aggregate = functools.partial(
    pl.kernel,
    out_type=jax.ShapeDtypeStruct((NSC * AGG_HALF, CH), jnp.float32),
    mesh=plsc.VectorSubcoreMesh(core_axis_name="c", subcore_axis_name="s"),
    scratch_types=(
        [pltpu.VMEM((B,), jnp.int32)] * 4
        + [pltpu.VMEM((K, CH), jnp.float32)] * 4
        + [pltpu.VMEM((K,), jnp.int32)] * 4
        + [pltpu.VMEM_SHARED((N, CH), jnp.float32)]
        + [pltpu.SemaphoreType.DMA] * 10
    ),
    compiler_params=pltpu.CompilerParams(use_tc_tiling_on_sc=False,
                                         needs_layout_passes=False),
)(_sc_body)


_R = 4000
_GN = 13
_RP = _R // 4
_RO = _R // 4


def _post_body(a0_ref, a1_ref, n4_ref, w0_ref, w1_ref, exp_ref, out_ref):
    dn = (((1,), (0,)), ((), ()))
    hp = lax.Precision.HIGHEST
    mm = (lax.dot_general(a0_ref[...], w0_ref[...], dn, precision=hp,
                          preferred_element_type=jnp.float32)
          + lax.dot_general(a1_ref[...], w1_ref[...], dn, precision=hp,
                            preferred_element_type=jnp.float32))
    nr = lax.dot_general(n4_ref[...], exp_ref[...], dn, precision=hp,
                         preferred_element_type=jnp.float32)
    out_ref[...] = nr * mm


def _tc_post(agg, norm, w):
    agg4 = agg.reshape(NSC * AGG_HALF // 4, 4 * CH)
    norm4 = norm.reshape(N // 4, 4)
    wb = jnp.zeros((NSC, 4 * CH, 4 * C), jnp.float32)
    for i in range(4):
        wb = wb.at[:, i * CH:(i + 1) * CH, i * C:(i + 1) * C].set(
            jnp.stack([w[:CH], w[CH:]]))
    exp = jnp.zeros((4, 4 * C), jnp.float32)
    for i in range(4):
        exp = exp.at[i, i * C:(i + 1) * C].set(1.0)
    out4 = pl.pallas_call(
        _post_body,
        grid=(_GN,),
        in_specs=[
            pl.BlockSpec((_RP, 4 * CH), lambda i: (i, 0)),
            pl.BlockSpec((_RP, 4 * CH), lambda i: (i + AGG_HALF // 4 // _RP, 0)),
            pl.BlockSpec((_RO, 4), lambda i: (i, 0)),
            pl.BlockSpec((4 * CH, 4 * C), lambda i: (0, 0)),
            pl.BlockSpec((4 * CH, 4 * C), lambda i: (1, 0)),
            pl.BlockSpec((4, 4 * C), lambda i: (0, 0)),
        ],
        out_specs=pl.BlockSpec((_RO, 4 * C), lambda i: (i, 0)),
        out_shape=jax.ShapeDtypeStruct((N // 4, 4 * C), jnp.float32),
    )(agg4, agg4, norm4, wb.reshape(NSC * 4 * CH, 4 * C), wb.reshape(
        NSC * 4 * CH, 4 * C), exp)
    return out4.reshape(N, C)


def kernel(x, sources, targets, norm, W):
    src = sources.astype(jnp.int32)
    tgt = targets.astype(jnp.int32)
    x2 = x.reshape(NSC * N, CH)
    agg = _sc_aggregate(x2, src, tgt)
    return _tc_post(agg, norm, W)


# speedup vs baseline: 3.6931x; 1.0574x over previous
import functools

import jax
import jax.numpy as jnp
from jax import lax
from jax.experimental import pallas as pl
from jax.experimental.pallas import tpu as pltpu
from jax.experimental.pallas import tpu_sc as plsc

N = 50000
C = 64
E = 800000

NSC = 2
NTILE = 16
CH = C // NSC
K = 80
CPB = 16
B = K * CPB
EPT = 49920
NBLK_LO = EPT // B
NBLK_HI = (E - (NTILE - 1) * EPT) // B
NBUF = 4
VPC = K // 16

STRIPE = 3120
LAST_STRIPE = N - (NTILE - 1) * STRIPE
NCI_LO = STRIPE // K
NCI_HI = LAST_STRIPE // K
AGG_HALF = 52000


def _sc_body(x2_hbm, src_hbm, tgt_hbm, out_hbm,
             sbufs0, sbuft0, sbufs1, sbuft1,
             rows0, rows1, rows2, rows3,
             sidx0, sidx1, sidx2, sidx3,
             acc,
             semg0, semg1, semg2, semg3,
             sems0, sems1, sems2, sems3,
             semi0, semi1):
    sc = lax.axis_index("c")
    tile = lax.axis_index("s")
    xbase = sc * AGG_HALF
    ebase = tile * EPT
    nblk = jnp.where(tile == NTILE - 1, NBLK_HI, NBLK_LO)

    sbufs = (sbufs0, sbufs1)
    sbuft = (sbuft0, sbuft1)
    semi = (semi0, semi1)
    rows = (rows0, rows1, rows2, rows3)
    sidx = (sidx0, sidx1, sidx2, sidx3)
    semg = (semg0, semg1, semg2, semg3)
    sems = (sems0, sems1, sems2, sems3)

    def _stage_start(blk, par):
        pltpu.async_copy(src_hbm.at[pl.ds(ebase + blk * B, B)],
                         sbufs[par], semi[par])
        pltpu.async_copy(tgt_hbm.at[pl.ds(ebase + blk * B, B)],
                         sbuft[par], semi[par])

    def _stage_wait(blk, par):
        pltpu.make_async_copy(src_hbm.at[pl.ds(ebase + blk * B, B)],
                              sbufs[par], semi[par]).wait()
        pltpu.make_async_copy(tgt_hbm.at[pl.ds(ebase + blk * B, B)],
                              sbuft[par], semi[par]).wait()

    def _gather_start(j, b, par):
        pltpu.async_copy(x2_hbm.at[sbufs[par].at[pl.ds(j * K, K)]],
                         rows[b], semg[b])

    def _gather_wait(j, b, par):
        pltpu.make_async_copy(x2_hbm.at[sbufs[par].at[pl.ds(j * K, K)]],
                              rows[b], semg[b]).wait()

    def _scatter_start(b):
        pltpu.async_copy(rows[b], acc.at[sidx[b]], sems[b], add=True)

    def _scatter_wait(b):
        pltpu.make_async_copy(rows[b], acc.at[sidx[b]], sems[b]).wait()

    _stage_start(0, 0)

    r0 = tile * STRIPE
    nci = jnp.where(tile == NTILE - 1, NCI_HI, NCI_LO)
    lane = lax.iota(jnp.int32, 16)

    def _ibuild(c, b):
        base = (r0 + c * K) * 2 + sc
        for v in range(VPC):
            sidx[b][pl.ds(v * 16, 16)] = base + (lane + v * 16) * 2

    def _igather_start(b):
        pltpu.async_copy(x2_hbm.at[sidx[b]], rows[b], semg[b])

    def _igather_wait(b):
        pltpu.make_async_copy(x2_hbm.at[sidx[b]], rows[b], semg[b]).wait()

    _ibuild(0, 0)
    _igather_start(0)

    def _ipair(p, carry):
        for par in range(2):
            c = p * 2 + par

            @pl.when(c < nci)
            def _():
                @pl.when(c + 1 < nci)
                def _():
                    _ibuild(c + 1, 1 - par)
                    _igather_start(1 - par)

                _igather_wait(par)
                pltpu.sync_copy(rows[par], acc.at[pl.ds(r0 + c * K, K)])
        return carry

    lax.fori_loop(0, (NCI_HI + 1) // 2, _ipair, 0)

    plsc.subcore_barrier()

    def _bias(par):
        def _bb(v, c):
            sl = pl.ds(v * 16, 16)
            sbufs[par][sl] = sbufs[par][sl] * 2 + sc
            return c

        lax.fori_loop(0, B // 16, _bb, 0, unroll=4)

    def _run_block(par):
        for jj in range(NBUF - 1):
            _gather_start(jj, jj, par)

        def _grp(gidx, c):
            for u in range(NBUF):
                j = gidx * NBUF + u
                bn = (u + NBUF - 1) % NBUF
                _gather_wait(j, u, par)
                for v in range(VPC):
                    sidx[u][pl.ds(v * 16, 16)] = (
                        sbuft[par][pl.ds(j * K + v * 16, 16)])
                _scatter_start(u)
                if u == 0:
                    @pl.when(gidx > 0)
                    def _():
                        _scatter_wait(bn)
                else:
                    _scatter_wait(bn)

                @pl.when(j < CPB - NBUF + 1)
                def _():
                    _gather_start(j + NBUF - 1, bn, par)
            return c

        lax.fori_loop(0, CPB // NBUF, _grp, 0)
        _scatter_wait(NBUF - 1)

    def _block_pair(p, carry):
        for par in range(2):
            blk = p * 2 + par

            @pl.when(blk < nblk)
            def _():
                _stage_wait(blk, par)

                @pl.when(blk + 1 < nblk)
                def _():
                    _stage_start(blk + 1, 1 - par)

                _bias(par)
                _run_block(par)
        return carry

    lax.fori_loop(0, (NBLK_HI + 1) // 2, _block_pair, 0)

    plsc.subcore_barrier()

    @pl.when(tile < NTILE - 1)
    def _():
        r0 = tile * STRIPE
        pltpu.sync_copy(acc.at[pl.ds(r0, STRIPE)],
                        out_hbm.at[pl.ds(xbase + r0, STRIPE)])

    @pl.when(tile == NTILE - 1)
    def _():
        r0 = (NTILE - 1) * STRIPE
        pltpu.sync_copy(acc.at[pl.ds(r0, LAST_STRIPE)],
                        out_hbm.at[pl.ds(xbase + r0, LAST_STRIPE)])


_sc_aggregate = functools.partial(
    pl.kernel,
    out_type=jax.ShapeDtypeStruct((NSC * AGG_HALF, CH), jnp.float32),
    mesh=plsc.VectorSubcoreMesh(core_axis_name="c", subcore_axis_name="s"),
    scratch_types=(
        [pltpu.VMEM((B,), jnp.int32)] * 4
        + [pltpu.VMEM((K, CH), jnp.float32)] * 4
        + [pltpu.VMEM((K,), jnp.int32)] * 4
        + [pltpu.VMEM_SHARED((N, CH), jnp.float32)]
        + [pltpu.SemaphoreType.DMA] * 10
    ),
    compiler_params=pltpu.CompilerParams(use_tc_tiling_on_sc=False,
                                         needs_layout_passes=False),
)(_sc_body)


_R = 4000
_GN = 13
_RP = _R // 4
_RO = _R // 4


def _post_body(a0_ref, a1_ref, n4_ref, w0_ref, w1_ref, exp_ref, out_ref):
    dn = (((1,), (0,)), ((), ()))
    hp = lax.Precision.DEFAULT
    mm = (lax.dot_general(a0_ref[...], w0_ref[...], dn, precision=hp,
                          preferred_element_type=jnp.float32)
          + lax.dot_general(a1_ref[...], w1_ref[...], dn, precision=hp,
                            preferred_element_type=jnp.float32))
    nr = lax.dot_general(n4_ref[...], exp_ref[...], dn, precision=hp,
                         preferred_element_type=jnp.float32)
    out_ref[...] = nr * mm


def _tc_post(agg, norm, w):
    agg4 = agg.reshape(NSC * AGG_HALF // 4, 4 * CH)
    norm4 = norm.reshape(N // 4, 4)
    wb = jnp.zeros((NSC, 4 * CH, 4 * C), jnp.float32)
    for i in range(4):
        wb = wb.at[:, i * CH:(i + 1) * CH, i * C:(i + 1) * C].set(
            jnp.stack([w[:CH], w[CH:]]))
    exp = jnp.zeros((4, 4 * C), jnp.float32)
    for i in range(4):
        exp = exp.at[i, i * C:(i + 1) * C].set(1.0)
    out4 = pl.pallas_call(
        _post_body,
        grid=(_GN,),
        in_specs=[
            pl.BlockSpec((_RP, 4 * CH), lambda i: (i, 0)),
            pl.BlockSpec((_RP, 4 * CH), lambda i: (i + AGG_HALF // 4 // _RP, 0)),
            pl.BlockSpec((_RO, 4), lambda i: (i, 0)),
            pl.BlockSpec((4 * CH, 4 * C), lambda i: (0, 0)),
            pl.BlockSpec((4 * CH, 4 * C), lambda i: (1, 0)),
            pl.BlockSpec((4, 4 * C), lambda i: (0, 0)),
        ],
        out_specs=pl.BlockSpec((_RO, 4 * C), lambda i: (i, 0)),
        out_shape=jax.ShapeDtypeStruct((N // 4, 4 * C), jnp.float32),
    )(agg4, agg4, norm4, wb.reshape(NSC * 4 * CH, 4 * C), wb.reshape(
        NSC * 4 * CH, 4 * C), exp)
    return out4.reshape(N, C)


def kernel(x, sources, targets, norm, W):
    src = sources.astype(jnp.int32)
    tgt = targets.astype(jnp.int32)
    x2 = x.reshape(NSC * N, CH)
    agg = _sc_aggregate(x2, src, tgt)
    return _tc_post(agg, norm, W)


# speedup vs baseline: 4.4354x; 1.2010x over previous
import functools

import jax
import jax.numpy as jnp
from jax import lax
from jax.experimental import pallas as pl
from jax.experimental.pallas import tpu as pltpu
from jax.experimental.pallas import tpu_sc as plsc

N = 50000
C = 64
E = 800000

NSC = 2
NTILE = 16
CH = C // NSC
K = 80
CPB = 16
B = K * CPB
EPT = 49920
NBLK_LO = EPT // B
NBLK_HI = (E - (NTILE - 1) * EPT) // B
NBUF = 8
VPC = K // 16

STRIPE = 3120
LAST_STRIPE = N - (NTILE - 1) * STRIPE
NCI_LO = STRIPE // K
NCI_HI = LAST_STRIPE // K
AGG_HALF = 52000


def _sc_body(x2_hbm, src_hbm, tgt_hbm, out_hbm,
             sbufs0, sbuft0, sbufs1, sbuft1,
             rows0, rows1, rows2, rows3,
             rows4, rows5, rows6, rows7,
             sidx0, sidx1, sidx2, sidx3,
             sidx4, sidx5, sidx6, sidx7,
             acc,
             semg0, semg1, semg2, semg3,
             semg4, semg5, semg6, semg7,
             sems0, sems1, sems2, sems3,
             sems4, sems5, sems6, sems7,
             semi0, semi1):
    sc = lax.axis_index("c")
    tile = lax.axis_index("s")
    xbase = sc * AGG_HALF
    ebase = tile * EPT
    nblk = jnp.where(tile == NTILE - 1, NBLK_HI, NBLK_LO)

    sbufs = (sbufs0, sbufs1)
    sbuft = (sbuft0, sbuft1)
    semi = (semi0, semi1)
    rows = (rows0, rows1, rows2, rows3, rows4, rows5, rows6, rows7)
    sidx = (sidx0, sidx1, sidx2, sidx3, sidx4, sidx5, sidx6, sidx7)
    semg = (semg0, semg1, semg2, semg3, semg4, semg5, semg6, semg7)
    sems = (sems0, sems1, sems2, sems3, sems4, sems5, sems6, sems7)

    def _stage_start(blk, par):
        pltpu.async_copy(src_hbm.at[pl.ds(ebase + blk * B, B)],
                         sbufs[par], semi[par])
        pltpu.async_copy(tgt_hbm.at[pl.ds(ebase + blk * B, B)],
                         sbuft[par], semi[par])

    def _stage_wait(blk, par):
        pltpu.make_async_copy(src_hbm.at[pl.ds(ebase + blk * B, B)],
                              sbufs[par], semi[par]).wait()
        pltpu.make_async_copy(tgt_hbm.at[pl.ds(ebase + blk * B, B)],
                              sbuft[par], semi[par]).wait()

    def _gather_start(j, b, par):
        pltpu.async_copy(x2_hbm.at[sbufs[par].at[pl.ds(j * K, K)]],
                         rows[b], semg[b])

    def _gather_wait(j, b, par):
        pltpu.make_async_copy(x2_hbm.at[sbufs[par].at[pl.ds(j * K, K)]],
                              rows[b], semg[b]).wait()

    def _scatter_start(b):
        pltpu.async_copy(rows[b], acc.at[sidx[b]], sems[b], add=True)

    def _scatter_wait(b):
        pltpu.make_async_copy(rows[b], acc.at[sidx[b]], sems[b]).wait()

    _stage_start(0, 0)

    r0 = tile * STRIPE
    nci = jnp.where(tile == NTILE - 1, NCI_HI, NCI_LO)
    lane = lax.iota(jnp.int32, 16)

    def _ibuild(c, b):
        base = (r0 + c * K) * 2 + sc
        for v in range(VPC):
            sidx[b][pl.ds(v * 16, 16)] = base + (lane + v * 16) * 2

    def _igather_start(b):
        pltpu.async_copy(x2_hbm.at[sidx[b]], rows[b], semg[b])

    def _igather_wait(b):
        pltpu.make_async_copy(x2_hbm.at[sidx[b]], rows[b], semg[b]).wait()

    _ibuild(0, 0)
    _igather_start(0)

    def _ipair(p, carry):
        for par in range(2):
            c = p * 2 + par

            @pl.when(c < nci)
            def _():
                @pl.when(c + 1 < nci)
                def _():
                    _ibuild(c + 1, 1 - par)
                    _igather_start(1 - par)

                _igather_wait(par)
                pltpu.sync_copy(rows[par], acc.at[pl.ds(r0 + c * K, K)])
        return carry

    lax.fori_loop(0, (NCI_HI + 1) // 2, _ipair, 0)

    plsc.subcore_barrier()

    def _bias(par):
        def _bb(v, c):
            sl = pl.ds(v * 16, 16)
            sbufs[par][sl] = sbufs[par][sl] * 2 + sc
            return c

        lax.fori_loop(0, B // 16, _bb, 0, unroll=4)

    def _run_block(par):
        for jj in range(NBUF - 1):
            _gather_start(jj, jj, par)

        def _grp(gidx, c):
            for u in range(NBUF):
                j = gidx * NBUF + u
                bn = (u + NBUF - 1) % NBUF
                _gather_wait(j, u, par)
                for v in range(VPC):
                    sidx[u][pl.ds(v * 16, 16)] = (
                        sbuft[par][pl.ds(j * K + v * 16, 16)])
                _scatter_start(u)
                if u == 0:
                    @pl.when(gidx > 0)
                    def _():
                        _scatter_wait(bn)
                else:
                    _scatter_wait(bn)

                @pl.when(j < CPB - NBUF + 1)
                def _():
                    _gather_start(j + NBUF - 1, bn, par)
            return c

        lax.fori_loop(0, CPB // NBUF, _grp, 0)
        _scatter_wait(NBUF - 1)

    def _block_pair(p, carry):
        for par in range(2):
            blk = p * 2 + par

            @pl.when(blk < nblk)
            def _():
                _stage_wait(blk, par)

                @pl.when(blk + 1 < nblk)
                def _():
                    _stage_start(blk + 1, 1 - par)

                _bias(par)
                _run_block(par)
        return carry

    lax.fori_loop(0, (NBLK_HI + 1) // 2, _block_pair, 0)

    plsc.subcore_barrier()

    @pl.when(tile < NTILE - 1)
    def _():
        r0 = tile * STRIPE
        pltpu.sync_copy(acc.at[pl.ds(r0, STRIPE)],
                        out_hbm.at[pl.ds(xbase + r0, STRIPE)])

    @pl.when(tile == NTILE - 1)
    def _():
        r0 = (NTILE - 1) * STRIPE
        pltpu.sync_copy(acc.at[pl.ds(r0, LAST_STRIPE)],
                        out_hbm.at[pl.ds(xbase + r0, LAST_STRIPE)])


_sc_aggregate = functools.partial(
    pl.kernel,
    out_type=jax.ShapeDtypeStruct((NSC * AGG_HALF, CH), jnp.float32),
    mesh=plsc.VectorSubcoreMesh(core_axis_name="c", subcore_axis_name="s"),
    scratch_types=(
        [pltpu.VMEM((B,), jnp.int32)] * 4
        + [pltpu.VMEM((K, CH), jnp.float32)] * 8
        + [pltpu.VMEM((K,), jnp.int32)] * 8
        + [pltpu.VMEM_SHARED((N, CH), jnp.float32)]
        + [pltpu.SemaphoreType.DMA] * 18
    ),
    compiler_params=pltpu.CompilerParams(use_tc_tiling_on_sc=False,
                                         needs_layout_passes=False),
)(_sc_body)


_R = 4000
_GN = 13
_RP = _R // 4
_RO = _R // 4


def _post_body(a0_ref, a1_ref, n4_ref, w0_ref, w1_ref, exp_ref, out_ref):
    dn = (((1,), (0,)), ((), ()))
    hp = lax.Precision.DEFAULT
    mm = (lax.dot_general(a0_ref[...], w0_ref[...], dn, precision=hp,
                          preferred_element_type=jnp.float32)
          + lax.dot_general(a1_ref[...], w1_ref[...], dn, precision=hp,
                            preferred_element_type=jnp.float32))
    nr = lax.dot_general(n4_ref[...], exp_ref[...], dn, precision=hp,
                         preferred_element_type=jnp.float32)
    out_ref[...] = nr * mm


def _tc_post(agg, norm, w):
    agg4 = agg.reshape(NSC * AGG_HALF // 4, 4 * CH)
    norm4 = norm.reshape(N // 4, 4)
    wb = jnp.zeros((NSC, 4 * CH, 4 * C), jnp.float32)
    for i in range(4):
        wb = wb.at[:, i * CH:(i + 1) * CH, i * C:(i + 1) * C].set(
            jnp.stack([w[:CH], w[CH:]]))
    exp = jnp.zeros((4, 4 * C), jnp.float32)
    for i in range(4):
        exp = exp.at[i, i * C:(i + 1) * C].set(1.0)
    out4 = pl.pallas_call(
        _post_body,
        grid=(_GN,),
        in_specs=[
            pl.BlockSpec((_RP, 4 * CH), lambda i: (i, 0)),
            pl.BlockSpec((_RP, 4 * CH), lambda i: (i + AGG_HALF // 4 // _RP, 0)),
            pl.BlockSpec((_RO, 4), lambda i: (i, 0)),
            pl.BlockSpec((4 * CH, 4 * C), lambda i: (0, 0)),
            pl.BlockSpec((4 * CH, 4 * C), lambda i: (1, 0)),
            pl.BlockSpec((4, 4 * C), lambda i: (0, 0)),
        ],
        out_specs=pl.BlockSpec((_RO, 4 * C), lambda i: (i, 0)),
        out_shape=jax.ShapeDtypeStruct((N // 4, 4 * C), jnp.float32),
    )(agg4, agg4, norm4, wb.reshape(NSC * 4 * CH, 4 * C), wb.reshape(
        NSC * 4 * CH, 4 * C), exp)
    return out4.reshape(N, C)


def kernel(x, sources, targets, norm, W):
    src = sources.astype(jnp.int32)
    tgt = targets.astype(jnp.int32)
    x2 = x.reshape(NSC * N, CH)
    agg = _sc_aggregate(x2, src, tgt)
    return _tc_post(agg, norm, W)
